# baseline (speedup 1.0000x reference)
import functools

import jax
import jax.numpy as jnp
from jax import lax
from jax.experimental import pallas as pl
from jax.experimental.pallas import tpu as pltpu
from jax.experimental.pallas import tpu_sc as plsc

D = 32
L = 16


@functools.cache
def _make_transpose(V: int):
    info = plsc.get_sparse_core_info()
    NC, NS = info.num_cores, info.num_subcores
    NW = NC * NS
    G = V // L
    GPW = -(-G // NW)
    CH = 512
    NCHF = (GPW * L) // CH
    TAIL = GPW * L - NCHF * CH

    mesh = plsc.VectorSubcoreMesh(core_axis_name="c", subcore_axis_name="s")

    @functools.partial(
        pl.kernel,
        mesh=mesh,
        compiler_params=pltpu.CompilerParams(
            use_tc_tiling_on_sc=False, needs_layout_passes=False
        ),
        out_type=jax.ShapeDtypeStruct((V, D), jnp.float32),
        scratch_types=[
            pltpu.VMEM((2, D, CH), jnp.float32),
            pltpu.VMEM((2, CH, D + 1), jnp.float32),
            pltpu.SemaphoreType.DMA,
            pltpu.SemaphoreType.DMA,
            pltpu.SemaphoreType.DMA,
        ],
    )
    def ka(tt_hbm, out_hbm, tin, tout, rsem0, rsem1, wsem):
        wid = lax.axis_index("s") * NC + lax.axis_index("c")
        vbase = jnp.minimum((G * wid) // NW, G - GPW) * L
        rsems = (rsem0, rsem1)
        vvecs = [g * L + lax.iota(jnp.int32, 16) for g in range(CH // L)]

        def fire_read(c, pb, n):
            pltpu.async_copy(
                tt_hbm.at[:, pl.ds(vbase + c * CH, n)],
                tin.at[pb, :, pl.ds(0, n)],
                rsems[pb],
            )

        def drain_read(pb, n):
            pltpu.make_async_copy(
                tt_hbm.at[:, pl.ds(0, n)], tin.at[pb, :, pl.ds(0, n)], rsems[pb]
            ).wait()

        def drain_write(pb, n):
            pltpu.make_async_copy(
                out_hbm.at[pl.ds(0, n)],
                tout.at[pb, pl.ds(0, n), pl.ds(0, D)],
                wsem,
            ).wait()

        def transpose(pb, ngroups):
            src = tin.at[pb]
            dst = tout.at[pb]

            @plsc.parallel_loop(0, D, 1, unroll=4)
            def _(e):
                ev = jnp.full((16,), 0, jnp.int32) + e
                for g in range(ngroups):
                    vals = src[e, pl.ds(g * L, L)]
                    plsc.store_scatter(dst, [vvecs[g], ev], vals)

        def step(c, pb, p, nxt_n):
            if nxt_n:
                fire_read(c + 1, 1 - pb, nxt_n)
            drain_read(pb, CH)

            @pl.when(p >= 1)
            def _():
                drain_write(pb, CH)

            transpose(pb, CH // L)
            pltpu.async_copy(
                tout.at[pb, :, pl.ds(0, D)],
                out_hbm.at[pl.ds(vbase + c * CH, CH)],
                wsem,
            )

        assert NCHF % 2 == 1 and TAIL > 0
        fire_read(0, 0, CH)

        def pair(p, carry):
            step(2 * p, 0, p, CH)
            step(2 * p + 1, 1, p, CH)
            return carry

        lax.fori_loop(0, NCHF // 2, pair, 0)
        step(NCHF - 1, 0, 1, TAIL)
        drain_read(1, TAIL)
        drain_write(1, CH)
        transpose(1, TAIL // L)
        pltpu.async_copy(
            tout.at[1, pl.ds(0, TAIL), pl.ds(0, D)],
            out_hbm.at[pl.ds(vbase + NCHF * CH, TAIL)],
            wsem,
        )
        drain_write(0, CH)
        drain_write(1, TAIL)

    return ka


@functools.cache
def _make_gather(BATCH: int, HIST: int):
    info = plsc.get_sparse_core_info()
    NC, NS = info.num_cores, info.num_subcores
    NW = NC * NS
    BPW = BATCH // NW
    NG = BPW // 128

    mesh = plsc.VectorSubcoreMesh(core_axis_name="c", subcore_axis_name="s")

    @functools.partial(
        pl.kernel,
        mesh=mesh,
        compiler_params=pltpu.CompilerParams(
            use_tc_tiling_on_sc=False, needs_layout_passes=False
        ),
        out_type=jax.ShapeDtypeStruct((HIST, D, BATCH), jnp.float32),
        scratch_types=[
            pltpu.VMEM((HIST, BPW), jnp.int32),
            pltpu.VMEM((2, BPW, D), jnp.float32),
            pltpu.VMEM((2, D, BPW + 1), jnp.float32),
            pltpu.SemaphoreType.DMA,
            pltpu.SemaphoreType.DMA,
            pltpu.SemaphoreType.DMA,
        ],
    )
    def k(table_hbm, idx_hbm, out_hbm, idx_v, gbuf, tbuf, gsem0, gsem1, wsem):
        wid = lax.axis_index("s") * NC + lax.axis_index("c")
        b0 = wid * BPW
        pltpu.sync_copy(idx_hbm.at[:, pl.ds(b0, BPW)], idx_v)

        gsems = (gsem0, gsem1)
        evecs = [eh * L + lax.iota(jnp.int32, 16) for eh in range(D // L)]

        def fire(h, gb):
            for j in range(NG):
                pltpu.async_copy(
                    table_hbm.at[idx_v.at[h, pl.ds(j * 128, 128)]],
                    gbuf.at[gb, pl.ds(j * 128, 128)],
                    gsems[gb],
                )

        def drain_gather(gb):
            for j in range(NG):
                pltpu.make_async_copy(
                    table_hbm.at[pl.ds(0, 128)],
                    gbuf.at[gb, pl.ds(j * 128, 128)],
                    gsems[gb],
                ).wait()

        def drain_write(gb):
            pltpu.make_async_copy(
                out_hbm.at[0, :, pl.ds(b0, BPW)],
                tbuf.at[gb, :, pl.ds(0, BPW)],
                wsem,
            ).wait()

        def step(h, gb, p):
            @pl.when(h + 1 < HIST)
            def _():
                fire(h + 1, 1 - gb)

            drain_gather(gb)

            @pl.when(p >= 1)
            def _():
                drain_write(gb)

            src = gbuf.at[gb]
            dst = tbuf.at[gb]

            @plsc.parallel_loop(0, BPW, 1, unroll=8)
            def tr_rows(b):
                bv = jnp.full((16,), 0, jnp.int32) + b
                for eh in range(D // L):
                    vals = src[b, pl.ds(eh * L, L)]
                    plsc.store_scatter(dst, [evecs[eh], bv], vals)
            pltpu.async_copy(
                tbuf.at[gb, :, pl.ds(0, BPW)],
                out_hbm.at[h, :, pl.ds(b0, BPW)],
                wsem,
            )

        fire(0, 0)

        def pair(p, carry):
            step(2 * p, 0, p)
            step(2 * p + 1, 1, p)
            return carry

        lax.fori_loop(0, HIST // 2, pair, 0)
        drain_write(0)
        drain_write(1)

    return k


def kernel(inputs, table):
    BATCH, HIST = inputs.shape
    idx_t = inputs.astype(jnp.int32).T
    tab_lin = _make_transpose(table.shape[0])(table.T)
    out = _make_gather(BATCH, HIST)(tab_lin, idx_t)
    return out.transpose(2, 0, 1)


# speedup vs baseline: 4.2489x; 4.2489x over previous
import functools

import jax
import jax.numpy as jnp
from jax import lax
from jax.experimental import pallas as pl
from jax.experimental.pallas import tpu as pltpu
from jax.experimental.pallas import tpu_sc as plsc

D = 32
L = 16


@functools.cache
def _make_gather(BATCH: int, HIST: int):
    info = plsc.get_sparse_core_info()
    NC, NS = info.num_cores, info.num_subcores
    NW = NC * NS
    BPW = BATCH // NW
    NG = BPW // 128

    mesh = plsc.VectorSubcoreMesh(core_axis_name="c", subcore_axis_name="s")

    @functools.partial(
        pl.kernel,
        mesh=mesh,
        compiler_params=pltpu.CompilerParams(
            use_tc_tiling_on_sc=False, needs_layout_passes=False
        ),
        out_type=jax.ShapeDtypeStruct((HIST, D, BATCH), jnp.float32),
        scratch_types=[
            pltpu.VMEM((HIST, BPW), jnp.int32),
            pltpu.VMEM((2, BPW, D), jnp.float32),
            pltpu.VMEM((2, D, BPW + 1), jnp.float32),
            pltpu.SemaphoreType.DMA,
            pltpu.SemaphoreType.DMA,
            pltpu.SemaphoreType.DMA,
        ],
    )
    def k(table_hbm, idx_hbm, out_hbm, idx_v, gbuf, tbuf, gsem0, gsem1, wsem):
        wid = lax.axis_index("s") * NC + lax.axis_index("c")
        b0 = wid * BPW
        pltpu.sync_copy(idx_hbm.at[:, pl.ds(b0, BPW)], idx_v)

        gsems = (gsem0, gsem1)
        evecs = [eh * L + lax.iota(jnp.int32, 16) for eh in range(D // L)]

        def fire(h, gb):
            for j in range(NG):
                pltpu.async_copy(
                    table_hbm.at[idx_v.at[h, pl.ds(j * 128, 128)]],
                    gbuf.at[gb, pl.ds(j * 128, 128)],
                    gsems[gb],
                )

        def drain_gather(gb):
            for j in range(NG):
                pltpu.make_async_copy(
                    table_hbm.at[pl.ds(0, 128)],
                    gbuf.at[gb, pl.ds(j * 128, 128)],
                    gsems[gb],
                ).wait()

        def drain_write(gb):
            pltpu.make_async_copy(
                out_hbm.at[0, :, pl.ds(b0, BPW)],
                tbuf.at[gb, :, pl.ds(0, BPW)],
                wsem,
            ).wait()

        def step(h, gb, p):
            @pl.when(h + 1 < HIST)
            def _():
                fire(h + 1, 1 - gb)

            drain_gather(gb)

            @pl.when(p >= 1)
            def _():
                drain_write(gb)

            src = gbuf.at[gb]
            dst = tbuf.at[gb]

            @plsc.parallel_loop(0, BPW, 1, unroll=8)
            def tr_rows(b):
                bv = jnp.full((16,), 0, jnp.int32) + b
                for eh in range(D // L):
                    vals = src[b, pl.ds(eh * L, L)]
                    plsc.store_scatter(dst, [evecs[eh], bv], vals)
            pltpu.async_copy(
                tbuf.at[gb, :, pl.ds(0, BPW)],
                out_hbm.at[h, :, pl.ds(b0, BPW)],
                wsem,
            )

        fire(0, 0)

        def pair(p, carry):
            step(2 * p, 0, p)
            step(2 * p + 1, 1, p)
            return carry

        lax.fori_loop(0, HIST // 2, pair, 0)
        drain_write(0)
        drain_write(1)

    return k


def kernel(inputs, table):
    BATCH, HIST = inputs.shape
    idx_t = inputs.astype(jnp.int32).T
    out = _make_gather(BATCH, HIST)(table, idx_t)
    return out.transpose(2, 0, 1)
